# baseline (speedup 1.0000x reference)
import jax
import jax.numpy as jnp
from jax import lax
from jax.experimental import pallas as pl
from jax.experimental.pallas import tpu as pltpu
from jax.experimental.pallas import tpu_sc as plsc

B = 32
NSAMP = 524288
F = 512
STRIDE = 256
NSEG = (NSAMP - F) // STRIDE + 1

NC = 2
NS = 16
NW = NC * NS

SEG_PER_W = 64
SUB = 32
SIG_CHUNK = SUB * STRIDE + (F - STRIDE)
TAIL = NSEG - (NW - 1) * SEG_PER_W - SUB
NOB = 3
LANES = 16


def _seg_body(
    sig_hbm, w_hbm, out_hbm, w_v, sig_v, out_v, sig_sem, out_sem, w_sem
):
    wid = lax.axis_index("s") * NC + lax.axis_index("c")
    seg_base = wid * SEG_PER_W
    is_tail = wid == NW - 1

    def sig_start(s0):
        return jnp.minimum(s0 * STRIDE, NSAMP - SIG_CHUNK)

    def sig_desc(b, h):
        s0 = seg_base + h * SUB
        return pltpu.make_async_copy(
            sig_hbm.at[b, pl.ds(sig_start(s0), SIG_CHUNK)],
            sig_v.at[h],
            sig_sem.at[h],
        )

    def slot_of(b, h):
        return lax.rem(2 * b + h, NOB)

    def out_desc_full(b, h):
        k = slot_of(b, h)
        s0 = seg_base + h * SUB
        return pltpu.make_async_copy(
            out_v.at[k], out_hbm.at[b, pl.ds(s0, SUB), :], out_sem.at[k]
        )

    def out_desc_tail(b):
        k = slot_of(b, 1)
        s0 = seg_base + SUB
        return pltpu.make_async_copy(
            out_v.at[k, pl.ds(0, TAIL), :],
            out_hbm.at[b, pl.ds(s0, TAIL), :],
            out_sem.at[k],
        )

    def start_out(b, h):
        if h == 0:
            out_desc_full(b, 0).start()
        else:

            @pl.when(jnp.logical_not(is_tail))
            def _s_full():
                out_desc_full(b, 1).start()

            @pl.when(is_tail)
            def _s_tail():
                out_desc_tail(b).start()

    def wait_out(b, h):
        if h == 0:
            out_desc_full(b, 0).wait()
        else:

            @pl.when(jnp.logical_not(is_tail))
            def _w_full():
                out_desc_full(b, 1).wait()

            @pl.when(is_tail)
            def _w_tail():
                out_desc_tail(b).wait()

    def compute(k, h, base, nseg):
        @plsc.parallel_loop(0, nseg, 1, unroll=2)
        def seg_body(i):
            for j in range(F // LANES):
                out_v[k, i, pl.ds(j * LANES, LANES)] = (
                    w_v[h * SUB + i, pl.ds(j * LANES, LANES)]
                    * sig_v[h, pl.ds(base + i * STRIDE + j * LANES, LANES)]
                )

    sig_desc(0, 0).start()
    sig_desc(0, 1).start()

    def w_desc_full():
        return pltpu.make_async_copy(
            w_hbm.at[pl.ds(seg_base, SEG_PER_W), :], w_v, w_sem
        )

    def w_desc_tail():
        return pltpu.make_async_copy(
            w_hbm.at[pl.ds(seg_base, SUB + TAIL), :],
            w_v.at[pl.ds(0, SUB + TAIL), :],
            w_sem,
        )

    @pl.when(jnp.logical_not(is_tail))
    def _w_stage_full():
        w_desc_full().start()
        w_desc_full().wait()

    @pl.when(is_tail)
    def _w_stage_tail():
        w_desc_tail().start()
        w_desc_tail().wait()

    def batch_body(b, carry):
        for h in (0, 1):
            s0 = seg_base + h * SUB
            sig_desc(b, h).wait()

            prev_b, prev_h = (b - 2, 1) if h == 0 else (b - 1, 0)

            @pl.when(prev_b >= 0)
            def _drain_prev():
                wait_out(prev_b, prev_h)

            base = s0 * STRIDE - sig_start(s0)
            k = slot_of(b, h)
            if h == 0:
                compute(k, 0, base, SUB)
            else:
                nseg = jnp.where(is_tail, TAIL, SUB)
                compute(k, 1, base, nseg)
            start_out(b, h)

            @pl.when(b + 1 < B)
            def _prefetch():
                sig_desc(b + 1, h).start()

        return carry

    lax.fori_loop(0, B, batch_body, 0)

    wait_out(B - 2, 1)
    wait_out(B - 1, 0)
    wait_out(B - 1, 1)


@jax.jit
def _segmentor(sig2d, weight_mat):
    mesh = plsc.VectorSubcoreMesh(core_axis_name="c", subcore_axis_name="s")
    out = pl.kernel(
        _seg_body,
        out_type=jax.ShapeDtypeStruct((B, NSEG, F), jnp.float32),
        mesh=mesh,
        compiler_params=pltpu.CompilerParams(use_tc_tiling_on_sc=False),
        scratch_types=[
            pltpu.VMEM((SEG_PER_W, F), jnp.float32),
            pltpu.VMEM((2, SIG_CHUNK), jnp.float32),
            pltpu.VMEM((NOB, SUB, F), jnp.float32),
            pltpu.SemaphoreType.DMA((2,)),
            pltpu.SemaphoreType.DMA((NOB,)),
            pltpu.SemaphoreType.DMA,
        ],
    )(sig2d, weight_mat)
    return out


def kernel(sig, idx_mat, weight_mat):
    sig2d = sig.reshape(B, NSAMP)
    out = _segmentor(sig2d, weight_mat)
    return out.reshape(B, 1, NSEG, F)


# speedup vs baseline: 1.0666x; 1.0666x over previous
import jax
import jax.numpy as jnp
from jax import lax
from jax.experimental import pallas as pl
from jax.experimental.pallas import tpu as pltpu
from jax.experimental.pallas import tpu_sc as plsc

B = 32
NSAMP = 524288
F = 512
STRIDE = 256
NSEG = (NSAMP - F) // STRIDE + 1

NC = 2
NS = 16
NW = NC * NS

SEG_PER_W = 64
SUB = 32
SIG_CHUNK = SUB * STRIDE + (F - STRIDE)
TAIL = NSEG - (NW - 1) * SEG_PER_W - SUB
LANES = 16


def _seg_body(
    sig_hbm, w_hbm, out_hbm, w_v, sig_v, out_v, sig_sem, out_sem, w_sem
):
    wid = lax.axis_index("s") * NC + lax.axis_index("c")
    seg_base = wid * SEG_PER_W
    is_tail = wid == NW - 1

    def sig_start(s0):
        return jnp.minimum(s0 * STRIDE, NSAMP - SIG_CHUNK)

    def sig_desc(b, h):
        s0 = seg_base + h * SUB
        return pltpu.make_async_copy(
            sig_hbm.at[b, pl.ds(sig_start(s0), SIG_CHUNK)],
            sig_v.at[h],
            sig_sem.at[h],
        )

    def out_desc_full(b, h):
        s0 = seg_base + h * SUB
        return pltpu.make_async_copy(
            out_v.at[h], out_hbm.at[b, pl.ds(s0, SUB), :], out_sem.at[h]
        )

    def out_desc_tail(b):
        s0 = seg_base + SUB
        return pltpu.make_async_copy(
            out_v.at[1, pl.ds(0, TAIL), :],
            out_hbm.at[b, pl.ds(s0, TAIL), :],
            out_sem.at[1],
        )

    HALF = STRIDE // LANES

    def compute(h, base, nseg):

        for q in range(HALF):
            v = sig_v[h, pl.ds(base + q * LANES, LANES)]
            out_v[h, 0, pl.ds(q * LANES, LANES)] = (
                w_v[h * SUB, pl.ds(q * LANES, LANES)] * v
            )

        @plsc.parallel_loop(1, nseg, 1, unroll=2)
        def row_body(r):
            for q in range(HALF):
                v = sig_v[h, pl.ds(base + r * STRIDE + q * LANES, LANES)]
                out_v[h, r, pl.ds(q * LANES, LANES)] = (
                    w_v[h * SUB + r, pl.ds(q * LANES, LANES)] * v
                )
                out_v[h, r - 1, pl.ds(STRIDE + q * LANES, LANES)] = (
                    w_v[h * SUB + r - 1, pl.ds(STRIDE + q * LANES, LANES)] * v
                )

        for q in range(HALF):
            v = sig_v[h, pl.ds(base + nseg * STRIDE + q * LANES, LANES)]
            out_v[h, nseg - 1, pl.ds(STRIDE + q * LANES, LANES)] = (
                w_v[h * SUB + nseg - 1, pl.ds(STRIDE + q * LANES, LANES)] * v
            )

    sig_desc(0, 0).start()
    sig_desc(0, 1).start()

    def w_desc_full():
        return pltpu.make_async_copy(
            w_hbm.at[pl.ds(seg_base, SEG_PER_W), :], w_v, w_sem
        )

    def w_desc_tail():
        return pltpu.make_async_copy(
            w_hbm.at[pl.ds(seg_base, SUB + TAIL), :],
            w_v.at[pl.ds(0, SUB + TAIL), :],
            w_sem,
        )

    @pl.when(jnp.logical_not(is_tail))
    def _w_stage_full():
        w_desc_full().start()
        w_desc_full().wait()

    @pl.when(is_tail)
    def _w_stage_tail():
        w_desc_tail().start()
        w_desc_tail().wait()

    def batch_body(b, carry):
        for h in (0, 1):
            s0 = seg_base + h * SUB
            sig_desc(b, h).wait()

            @pl.when(b >= 1)
            def _drain_prev():
                if h == 0:
                    out_desc_full(b - 1, 0).wait()
                else:

                    @pl.when(jnp.logical_not(is_tail))
                    def _w_full():
                        out_desc_full(b - 1, 1).wait()

                    @pl.when(is_tail)
                    def _w_tail():
                        out_desc_tail(b - 1).wait()

            base = s0 * STRIDE - sig_start(s0)
            if h == 0:
                compute(0, base, SUB)
                out_desc_full(b, 0).start()
            else:
                nseg = jnp.where(is_tail, TAIL, SUB)
                compute(1, base, nseg)

                @pl.when(jnp.logical_not(is_tail))
                def _s_full():
                    out_desc_full(b, 1).start()

                @pl.when(is_tail)
                def _s_tail():
                    out_desc_tail(b).start()

            @pl.when(b + 1 < B)
            def _prefetch():
                sig_desc(b + 1, h).start()

        return carry

    lax.fori_loop(0, B, batch_body, 0)

    out_desc_full(B - 1, 0).wait()

    @pl.when(jnp.logical_not(is_tail))
    def _d_full():
        out_desc_full(B - 1, 1).wait()

    @pl.when(is_tail)
    def _d_tail():
        out_desc_tail(B - 1).wait()


@jax.jit
def _segmentor(sig2d, w_pad):
    mesh = plsc.VectorSubcoreMesh(core_axis_name="c", subcore_axis_name="s")
    out = pl.kernel(
        _seg_body,
        out_type=jax.ShapeDtypeStruct((B, NSEG, F), jnp.float32),
        mesh=mesh,
        compiler_params=pltpu.CompilerParams(use_tc_tiling_on_sc=False),
        scratch_types=[
            pltpu.VMEM((SEG_PER_W, F), jnp.float32),
            pltpu.VMEM((2, SIG_CHUNK), jnp.float32),
            pltpu.VMEM((2, SUB, F), jnp.float32),
            pltpu.SemaphoreType.DMA((2,)),
            pltpu.SemaphoreType.DMA((2,)),
            pltpu.SemaphoreType.DMA,
        ],
    )(sig2d, w_pad)
    return out


def kernel(sig, idx_mat, weight_mat):
    sig2d = sig.reshape(B, NSAMP)
    out = _segmentor(sig2d, weight_mat)
    return out.reshape(B, 1, NSEG, F)
